# Initial kernel scaffold; baseline (speedup 1.0000x reference)
import jax
import jax.numpy as jnp
from jax.experimental import pallas as pl


def kernel(a, p, embeddings, W, b):
    raise NotImplementedError("write your pallas kernel here")


# speedup vs baseline: 1.3467x; 1.3467x over previous
import functools

import jax
import jax.numpy as jnp
from jax import lax
from jax.experimental import pallas as pl
from jax.experimental.pallas import tpu as pltpu
from jax.experimental.pallas import tpu_sc as plsc

VOCAB = 1000000
DIM = 32
BATCH = 16384

NC = 2
NS = 16
L = 16
NW = NC * NS
B_PER_W = BATCH // NW
CHUNK = 128
N_CHUNKS = B_PER_W // CHUNK
N_BLOCKS = B_PER_W // L
BLOCKS_PER_GROUP = 4
N_GROUPS = N_BLOCKS // BLOCKS_PER_GROUP


def _sc_body(a_hbm, p_hbm, emb_hbm, wbb_hbm, out_hbm,
             idx_va, idx_vp, rows_va, rows_vp, wbb_v, out_v, sem):
    wid = lax.axis_index("s") * NC + lax.axis_index("c")
    base = wid * B_PER_W

    for c in range(N_CHUNKS):
        pltpu.sync_copy(a_hbm.at[pl.ds(base + c * CHUNK, CHUNK)], idx_va.at[c])
        pltpu.sync_copy(p_hbm.at[pl.ds(base + c * CHUNK, CHUNK)], idx_vp.at[c])
    pltpu.sync_copy(wbb_hbm, wbb_v)

    copies = []
    for c in range(N_CHUNKS):
        copies.append(pltpu.async_copy(
            emb_hbm.at[idx_va.at[c]],
            rows_va.at[pl.ds(c * CHUNK, CHUNK), :], sem))
        copies.append(pltpu.async_copy(
            emb_hbm.at[idx_vp.at[c]],
            rows_vp.at[pl.ds(c * CHUNK, CHUNK), :], sem))
    for cp in copies:
        cp.wait()

    iota = lax.iota(jnp.int32, L)
    bb0 = wbb_v[pl.ds(2 * DIM * 2 * L, L)]
    bb1 = wbb_v[pl.ds(2 * DIM * 2 * L + L, L)]

    def group_body(g, _):
        row0 = g * (BLOCKS_PER_GROUP * L)
        row_ids = [iota + (row0 + blk * L) for blk in range(BLOCKS_PER_GROUP)]
        acc = [[bb0, bb1] for _ in range(BLOCKS_PER_GROUP)]
        for d in range(2 * DIM):
            w0 = wbb_v[pl.ds((d * 2) * L, L)]
            w1 = wbb_v[pl.ds((d * 2 + 1) * L, L)]
            src = rows_va if d < DIM else rows_vp
            col = jnp.full((L,), d % DIM, jnp.int32)
            for blk in range(BLOCKS_PER_GROUP):
                v = plsc.load_gather(src, [row_ids[blk], col])
                acc[blk][0] = acc[blk][0] + v * w0
                acc[blk][1] = acc[blk][1] + v * w1
        for blk in range(BLOCKS_PER_GROUP):
            oidx = iota * 2 + (row0 + blk * L) * 2
            plsc.store_scatter(out_v, [oidx], acc[blk][0])
            plsc.store_scatter(out_v, [oidx + 1], acc[blk][1])
        return ()

    lax.fori_loop(0, N_GROUPS, group_body, ())

    pltpu.sync_copy(out_v, out_hbm.at[pl.ds(wid * (2 * B_PER_W), 2 * B_PER_W)])


@jax.jit
def _linear_model_sc(a, p, embeddings, wbb):
    mesh = plsc.VectorSubcoreMesh(
        core_axis_name="c", subcore_axis_name="s",
        num_cores=NC, num_subcores=NS)
    f = pl.kernel(
        _sc_body,
        out_type=jax.ShapeDtypeStruct((BATCH * 2,), jnp.float32),
        mesh=mesh,
        scratch_types=[
            pltpu.VMEM((N_CHUNKS, CHUNK), jnp.int32),
            pltpu.VMEM((N_CHUNKS, CHUNK), jnp.int32),
            pltpu.VMEM((B_PER_W, DIM), jnp.float32),
            pltpu.VMEM((B_PER_W, DIM), jnp.float32),
            pltpu.VMEM((2 * DIM * 2 * L + 2 * L,), jnp.float32),
            pltpu.VMEM((2 * B_PER_W,), jnp.float32),
            pltpu.SemaphoreType.DMA,
        ],
        compiler_params=pltpu.CompilerParams(
            needs_layout_passes=False, use_tc_tiling_on_sc=False),
    )
    return f(a, p, embeddings, wbb)


def kernel(a, p, embeddings, W, b):
    wb = jnp.broadcast_to(W.reshape(2 * DIM * 2, 1), (2 * DIM * 2, L))
    bb = jnp.broadcast_to(b.reshape(2, 1), (2, L))
    wbb = jnp.concatenate([wb.reshape(-1), bb.reshape(-1)])
    out_flat = _linear_model_sc(a.astype(jnp.int32), p.astype(jnp.int32),
                                embeddings, wbb)
    return out_flat.reshape(BATCH, 2)


# speedup vs baseline: 4.8212x; 3.5800x over previous
import functools

import jax
import jax.numpy as jnp
from jax import lax
from jax.experimental import pallas as pl
from jax.experimental.pallas import tpu as pltpu
from jax.experimental.pallas import tpu_sc as plsc

VOCAB = 1000000
DIM = 32
BATCH = 16384
ROWS_PER_LINE = 4
SPLIT = 1 << 18

NC = 2
NS = 16
L = 16
NW = NC * NS
B_PER_W = BATCH // NW
CHUNK = 128
N_CHUNKS = B_PER_W // CHUNK
BLOCKS_PER_GROUP = 4
GROUPS_PER_CHUNK = CHUNK // (BLOCKS_PER_GROUP * L)

WB_SIZE = 2 * DIM * 2 * L + 2 * L

REPACK_BL = 8192
MAX_IN_BLOCK = VOCAB // REPACK_BL


def _repack_body(x0_ref, x1_ref, x2_ref, x3_ref, o_ref):
    y = jnp.concatenate(
        [r[...].reshape(DIM, REPACK_BL) for r in (x0_ref, x1_ref, x2_ref,
                                                  x3_ref)], axis=0)
    o_ref[...] = y.T


def _repack(emb3):
    def in_spec(j2):
        return pl.BlockSpec(
            (ROWS_PER_LINE, 8, REPACK_BL),
            lambda j, j2=j2: (0, 0,
                              jnp.minimum(j2 * (SPLIT // REPACK_BL) + j,
                                          MAX_IN_BLOCK)))
    return pl.pallas_call(
        _repack_body,
        grid=(SPLIT // REPACK_BL,),
        in_specs=[in_spec(j2) for j2 in range(ROWS_PER_LINE)],
        out_specs=pl.BlockSpec((REPACK_BL, ROWS_PER_LINE * DIM),
                               lambda j: (j, 0)),
        out_shape=jax.ShapeDtypeStruct((SPLIT, ROWS_PER_LINE * DIM),
                                       jnp.float32),
    )(emb3, emb3, emb3, emb3)


def _sc_body(a_hbm, p_hbm, emb_hbm, wbb_hbm, out_hbm,
             idx_raw, idx_line, rows_va, rows_vp, wbb_v, out_v, sem):
    wid = lax.axis_index("s") * NC + lax.axis_index("c")
    base = wid * B_PER_W

    for c in range(N_CHUNKS):
        pltpu.sync_copy(a_hbm.at[pl.ds(base + c * CHUNK, CHUNK)],
                        idx_raw.at[c])
        pltpu.sync_copy(p_hbm.at[pl.ds(base + c * CHUNK, CHUNK)],
                        idx_raw.at[N_CHUNKS + c])
    pltpu.sync_copy(wbb_hbm, wbb_v)

    lmask = jnp.full((L,), SPLIT - 1, jnp.int32)
    for r in range(2 * N_CHUNKS):
        for s in range(CHUNK // L):
            v = idx_raw[r, pl.ds(s * L, L)]
            idx_line[r, pl.ds(s * L, L)] = lax.bitwise_and(v, lmask)

    iota = lax.iota(jnp.int32, L)
    bb0 = wbb_v[pl.ds(2 * DIM * 2 * L, L)]
    bb1 = wbb_v[pl.ds(2 * DIM * 2 * L + L, L)]
    pmask = jnp.full((L,), 96, jnp.int32)

    def chunk_body(c, _):
        cp_a = pltpu.async_copy(emb_hbm.at[idx_line.at[c]], rows_va, sem)
        cp_p = pltpu.async_copy(emb_hbm.at[idx_line.at[N_CHUNKS + c]],
                                rows_vp, sem)
        cp_a.wait()
        cp_p.wait()

        def group_body(g, _):
            blk0 = g * BLOCKS_PER_GROUP
            rows, pha, php = [], [], []
            for blk in range(BLOCKS_PER_GROUP):
                rows.append(iota + (blk0 + blk) * L)
                ra = idx_raw[c, pl.ds((blk0 + blk) * L, L)]
                rp = idx_raw[N_CHUNKS + c, pl.ds((blk0 + blk) * L, L)]
                pha.append(lax.bitwise_and(
                    lax.shift_right_logical(ra, 13), pmask))
                php.append(lax.bitwise_and(
                    lax.shift_right_logical(rp, 13), pmask))
            acc = [[bb0, bb1] for _ in range(BLOCKS_PER_GROUP)]
            for d in range(2 * DIM):
                w0 = wbb_v[pl.ds((d * 2) * L, L)]
                w1 = wbb_v[pl.ds((d * 2 + 1) * L, L)]
                src = rows_va if d < DIM else rows_vp
                ph = pha if d < DIM else php
                dd = d % DIM
                for blk in range(BLOCKS_PER_GROUP):
                    col = ph[blk] + dd
                    v = plsc.load_gather(src, [rows[blk], col])
                    acc[blk][0] = acc[blk][0] + v * w0
                    acc[blk][1] = acc[blk][1] + v * w1
            row0 = c * CHUNK + blk0 * L
            for blk in range(BLOCKS_PER_GROUP):
                oidx = iota * 2 + (row0 + blk * L) * 2
                plsc.store_scatter(out_v, [oidx], acc[blk][0])
                plsc.store_scatter(out_v, [oidx + 1], acc[blk][1])
            return ()

        lax.fori_loop(0, GROUPS_PER_CHUNK, group_body, ())
        return ()

    lax.fori_loop(0, N_CHUNKS, chunk_body, ())

    pltpu.sync_copy(out_v, out_hbm.at[pl.ds(wid * (2 * B_PER_W), 2 * B_PER_W)])


@jax.jit
def _linear_model_sc(a, p, embeddings, wbb):
    emb_lines = _repack(embeddings.T.reshape(ROWS_PER_LINE, 8, VOCAB))
    mesh = plsc.VectorSubcoreMesh(
        core_axis_name="c", subcore_axis_name="s",
        num_cores=NC, num_subcores=NS)
    f = pl.kernel(
        _sc_body,
        out_type=jax.ShapeDtypeStruct((BATCH * 2,), jnp.float32),
        mesh=mesh,
        scratch_types=[
            pltpu.VMEM((2 * N_CHUNKS, CHUNK), jnp.int32),
            pltpu.VMEM((2 * N_CHUNKS, CHUNK), jnp.int32),
            pltpu.VMEM((CHUNK, ROWS_PER_LINE * DIM), jnp.float32),
            pltpu.VMEM((CHUNK, ROWS_PER_LINE * DIM), jnp.float32),
            pltpu.VMEM((WB_SIZE,), jnp.float32),
            pltpu.VMEM((2 * B_PER_W,), jnp.float32),
            pltpu.SemaphoreType.DMA,
        ],
        compiler_params=pltpu.CompilerParams(needs_layout_passes=False),
    )
    return f(a, p, emb_lines, wbb)


def kernel(a, p, embeddings, W, b):
    wb = jnp.broadcast_to(W.reshape(2 * DIM * 2, 1), (2 * DIM * 2, L))
    bb = jnp.broadcast_to(b.reshape(2, 1), (2, L))
    wbb = jnp.concatenate([wb.reshape(-1), bb.reshape(-1)])
    out_flat = _linear_model_sc(a.astype(jnp.int32), p.astype(jnp.int32),
                                embeddings, wbb)
    return out_flat.reshape(BATCH, 2)


# speedup vs baseline: 4.9401x; 1.0247x over previous
import functools

import jax
import jax.numpy as jnp
from jax import lax
from jax.experimental import pallas as pl
from jax.experimental.pallas import tpu as pltpu
from jax.experimental.pallas import tpu_sc as plsc

VOCAB = 1000000
DIM = 32
BATCH = 16384
ROWS_PER_LINE = 4
SPLIT = 1 << 18

NC = 2
NS = 16
L = 16
NW = NC * NS
B_PER_W = BATCH // NW
CHUNK = 128
N_CHUNKS = B_PER_W // CHUNK
BLOCKS_PER_GROUP = 4
GROUPS_PER_CHUNK = CHUNK // (BLOCKS_PER_GROUP * L)

WB_SIZE = 2 * DIM * 2 * L + 2 * L

REPACK_BL = 8192
MAX_IN_BLOCK = VOCAB // REPACK_BL


def _repack_body(x0_ref, x1_ref, x2_ref, x3_ref, o_ref):
    y = jnp.concatenate(
        [r[...].reshape(DIM, REPACK_BL) for r in (x0_ref, x1_ref, x2_ref,
                                                  x3_ref)], axis=0)
    o_ref[...] = y.T


def _repack(emb3):
    def in_spec(j2):
        return pl.BlockSpec(
            (ROWS_PER_LINE, 8, REPACK_BL),
            lambda j, j2=j2: (0, 0,
                              jnp.minimum(j2 * (SPLIT // REPACK_BL) + j,
                                          MAX_IN_BLOCK)))
    return pl.pallas_call(
        _repack_body,
        grid=(SPLIT // REPACK_BL,),
        in_specs=[in_spec(j2) for j2 in range(ROWS_PER_LINE)],
        out_specs=pl.BlockSpec((REPACK_BL, ROWS_PER_LINE * DIM),
                               lambda j: (j, 0)),
        out_shape=jax.ShapeDtypeStruct((SPLIT, ROWS_PER_LINE * DIM),
                                       jnp.float32),
    )(emb3, emb3, emb3, emb3)


def _sc_body(a_hbm, p_hbm, emb_hbm, wbb_hbm, out_hbm,
             idx_raw, idx_line, rows_va, rows_vp, wbb_v, out_v, sem):
    wid = lax.axis_index("s") * NC + lax.axis_index("c")
    base = wid * B_PER_W

    for c in range(N_CHUNKS):
        pltpu.sync_copy(a_hbm.at[pl.ds(base + c * CHUNK, CHUNK)],
                        idx_raw.at[c])
        pltpu.sync_copy(p_hbm.at[pl.ds(base + c * CHUNK, CHUNK)],
                        idx_raw.at[N_CHUNKS + c])
    pltpu.sync_copy(wbb_hbm, wbb_v)

    lmask = jnp.full((L,), SPLIT - 1, jnp.int32)
    for r in range(2 * N_CHUNKS):
        for s in range(CHUNK // L):
            v = idx_raw[r, pl.ds(s * L, L)]
            idx_line[r, pl.ds(s * L, L)] = lax.bitwise_and(v, lmask)

    iota = lax.iota(jnp.int32, L)
    bb0 = wbb_v[pl.ds(2 * DIM * 2 * L, L)]
    bb1 = wbb_v[pl.ds(2 * DIM * 2 * L + L, L)]
    pmask = jnp.full((L,), 96, jnp.int32)

    def fire(c, buf):
        return (pltpu.async_copy(emb_hbm.at[idx_line.at[c]],
                                 rows_va.at[buf], sem),
                pltpu.async_copy(emb_hbm.at[idx_line.at[N_CHUNKS + c]],
                                 rows_vp.at[buf], sem))

    inflight = fire(0, 0)
    for c in range(N_CHUNKS):
        buf = c % 2
        cps = inflight
        if c + 1 < N_CHUNKS:
            inflight = fire(c + 1, (c + 1) % 2)
        for cp in cps:
            cp.wait()

        def group_body(g, _, c=c, buf=buf):
            blk0 = g * BLOCKS_PER_GROUP
            rows, pha, php = [], [], []
            for blk in range(BLOCKS_PER_GROUP):
                rows.append(iota + (blk0 + blk) * L)
                ra = idx_raw[c, pl.ds((blk0 + blk) * L, L)]
                rp = idx_raw[N_CHUNKS + c, pl.ds((blk0 + blk) * L, L)]
                pha.append(lax.bitwise_and(
                    lax.shift_right_logical(ra, 13), pmask))
                php.append(lax.bitwise_and(
                    lax.shift_right_logical(rp, 13), pmask))
            acc = [[bb0, bb1] for _ in range(BLOCKS_PER_GROUP)]
            for d in range(2 * DIM):
                w0 = wbb_v[pl.ds((d * 2) * L, L)]
                w1 = wbb_v[pl.ds((d * 2 + 1) * L, L)]
                src = rows_va.at[buf] if d < DIM else rows_vp.at[buf]
                ph = pha if d < DIM else php
                dd = d % DIM
                for blk in range(BLOCKS_PER_GROUP):
                    col = ph[blk] + dd
                    v = plsc.load_gather(src, [rows[blk], col])
                    acc[blk][0] = acc[blk][0] + v * w0
                    acc[blk][1] = acc[blk][1] + v * w1
            row0 = c * CHUNK + blk0 * L
            for blk in range(BLOCKS_PER_GROUP):
                oidx = iota * 2 + (row0 + blk * L) * 2
                plsc.store_scatter(out_v, [oidx], acc[blk][0])
                plsc.store_scatter(out_v, [oidx + 1], acc[blk][1])
            return ()

        lax.fori_loop(0, GROUPS_PER_CHUNK, group_body, ())

    pltpu.sync_copy(out_v, out_hbm.at[pl.ds(wid * (2 * B_PER_W), 2 * B_PER_W)])


@jax.jit
def _linear_model_sc(a, p, embeddings, wbb):
    emb_lines = _repack(embeddings.T.reshape(ROWS_PER_LINE, 8, VOCAB))
    mesh = plsc.VectorSubcoreMesh(
        core_axis_name="c", subcore_axis_name="s",
        num_cores=NC, num_subcores=NS)
    f = pl.kernel(
        _sc_body,
        out_type=jax.ShapeDtypeStruct((BATCH * 2,), jnp.float32),
        mesh=mesh,
        scratch_types=[
            pltpu.VMEM((2 * N_CHUNKS, CHUNK), jnp.int32),
            pltpu.VMEM((2 * N_CHUNKS, CHUNK), jnp.int32),
            pltpu.VMEM((2, CHUNK, ROWS_PER_LINE * DIM), jnp.float32),
            pltpu.VMEM((2, CHUNK, ROWS_PER_LINE * DIM), jnp.float32),
            pltpu.VMEM((WB_SIZE,), jnp.float32),
            pltpu.VMEM((2 * B_PER_W,), jnp.float32),
            pltpu.SemaphoreType.DMA,
        ],
        compiler_params=pltpu.CompilerParams(needs_layout_passes=False),
    )
    return f(a, p, emb_lines, wbb)


def kernel(a, p, embeddings, W, b):
    wb = jnp.broadcast_to(W.reshape(2 * DIM * 2, 1), (2 * DIM * 2, L))
    bb = jnp.broadcast_to(b.reshape(2, 1), (2, L))
    wbb = jnp.concatenate([wb.reshape(-1), bb.reshape(-1)])
    out_flat = _linear_model_sc(a.astype(jnp.int32), p.astype(jnp.int32),
                                embeddings, wbb)
    return out_flat.reshape(BATCH, 2)


# speedup vs baseline: 5.0061x; 1.0134x over previous
import functools

import jax
import jax.numpy as jnp
from jax import lax
from jax.experimental import pallas as pl
from jax.experimental.pallas import tpu as pltpu
from jax.experimental.pallas import tpu_sc as plsc

VOCAB = 1000000
DIM = 32
BATCH = 16384
ROWS_PER_LINE = 4
SPLIT = 1 << 18

NC = 2
NS = 16
L = 16
NW = NC * NS
B_PER_W = BATCH // NW
CHUNK = 128
N_CHUNKS = B_PER_W // CHUNK
BLOCKS_PER_GROUP = 4
GROUPS_PER_CHUNK = CHUNK // (BLOCKS_PER_GROUP * L)

WB_SIZE = 2 * DIM * 2 * L + 2 * L

REPACK_BL = 16384
MAX_IN_BLOCK = VOCAB // REPACK_BL


def _repack_body(x0_ref, x1_ref, x2_ref, x3_ref, o_ref):
    y = jnp.concatenate(
        [r[...].reshape(DIM, REPACK_BL) for r in (x0_ref, x1_ref, x2_ref,
                                                  x3_ref)], axis=0)
    o_ref[...] = y.T


def _repack(emb3):
    def in_spec(j2):
        return pl.BlockSpec(
            (ROWS_PER_LINE, 8, REPACK_BL),
            lambda j, j2=j2: (0, 0,
                              jnp.minimum(j2 * (SPLIT // REPACK_BL) + j,
                                          MAX_IN_BLOCK)))
    return pl.pallas_call(
        _repack_body,
        grid=(SPLIT // REPACK_BL,),
        in_specs=[in_spec(j2) for j2 in range(ROWS_PER_LINE)],
        out_specs=pl.BlockSpec((REPACK_BL, ROWS_PER_LINE * DIM),
                               lambda j: (j, 0)),
        out_shape=jax.ShapeDtypeStruct((SPLIT, ROWS_PER_LINE * DIM),
                                       jnp.float32),
    )(emb3, emb3, emb3, emb3)


def _sc_body(a_hbm, p_hbm, emb_hbm, wbb_hbm, out_hbm,
             idx_raw, idx_line, rows_va, rows_vp, wbb_v, out_v, sem):
    wid = lax.axis_index("s") * NC + lax.axis_index("c")
    base = wid * B_PER_W

    for c in range(N_CHUNKS):
        pltpu.sync_copy(a_hbm.at[pl.ds(base + c * CHUNK, CHUNK)],
                        idx_raw.at[c])
        pltpu.sync_copy(p_hbm.at[pl.ds(base + c * CHUNK, CHUNK)],
                        idx_raw.at[N_CHUNKS + c])
    pltpu.sync_copy(wbb_hbm, wbb_v)

    lmask = jnp.full((L,), SPLIT - 1, jnp.int32)
    for r in range(2 * N_CHUNKS):
        for s in range(CHUNK // L):
            v = idx_raw[r, pl.ds(s * L, L)]
            idx_line[r, pl.ds(s * L, L)] = lax.bitwise_and(v, lmask)

    iota = lax.iota(jnp.int32, L)
    bb0 = wbb_v[pl.ds(2 * DIM * 2 * L, L)]
    bb1 = wbb_v[pl.ds(2 * DIM * 2 * L + L, L)]
    pmask = jnp.full((L,), 96, jnp.int32)

    def fire(c, buf):
        return (pltpu.async_copy(emb_hbm.at[idx_line.at[c]],
                                 rows_va.at[buf], sem),
                pltpu.async_copy(emb_hbm.at[idx_line.at[N_CHUNKS + c]],
                                 rows_vp.at[buf], sem))

    inflight = fire(0, 0)
    for c in range(N_CHUNKS):
        buf = c % 2
        cps = inflight
        if c + 1 < N_CHUNKS:
            inflight = fire(c + 1, (c + 1) % 2)
        for cp in cps:
            cp.wait()

        def group_body(g, _, c=c, buf=buf):
            blk0 = g * BLOCKS_PER_GROUP
            rows, pha, php = [], [], []
            for blk in range(BLOCKS_PER_GROUP):
                rows.append(iota + (blk0 + blk) * L)
                ra = idx_raw[c, pl.ds((blk0 + blk) * L, L)]
                rp = idx_raw[N_CHUNKS + c, pl.ds((blk0 + blk) * L, L)]
                pha.append(lax.bitwise_and(
                    lax.shift_right_logical(ra, 13), pmask))
                php.append(lax.bitwise_and(
                    lax.shift_right_logical(rp, 13), pmask))
            acc = [[bb0, bb1] for _ in range(BLOCKS_PER_GROUP)]
            for d in range(2 * DIM):
                w0 = wbb_v[pl.ds((d * 2) * L, L)]
                w1 = wbb_v[pl.ds((d * 2 + 1) * L, L)]
                src = rows_va.at[buf] if d < DIM else rows_vp.at[buf]
                ph = pha if d < DIM else php
                dd = d % DIM
                for blk in range(BLOCKS_PER_GROUP):
                    col = ph[blk] + dd
                    v = plsc.load_gather(src, [rows[blk], col])
                    acc[blk][0] = acc[blk][0] + v * w0
                    acc[blk][1] = acc[blk][1] + v * w1
            row0 = c * CHUNK + blk0 * L
            for blk in range(BLOCKS_PER_GROUP):
                oidx = iota * 2 + (row0 + blk * L) * 2
                plsc.store_scatter(out_v, [oidx], acc[blk][0])
                plsc.store_scatter(out_v, [oidx + 1], acc[blk][1])
            return ()

        lax.fori_loop(0, GROUPS_PER_CHUNK, group_body, ())

    pltpu.sync_copy(out_v, out_hbm.at[pl.ds(wid * (2 * B_PER_W), 2 * B_PER_W)])


@jax.jit
def _linear_model_sc(a, p, embeddings, wbb):
    emb_lines = _repack(embeddings.T.reshape(ROWS_PER_LINE, 8, VOCAB))
    mesh = plsc.VectorSubcoreMesh(
        core_axis_name="c", subcore_axis_name="s",
        num_cores=NC, num_subcores=NS)
    f = pl.kernel(
        _sc_body,
        out_type=jax.ShapeDtypeStruct((BATCH * 2,), jnp.float32),
        mesh=mesh,
        scratch_types=[
            pltpu.VMEM((2 * N_CHUNKS, CHUNK), jnp.int32),
            pltpu.VMEM((2 * N_CHUNKS, CHUNK), jnp.int32),
            pltpu.VMEM((2, CHUNK, ROWS_PER_LINE * DIM), jnp.float32),
            pltpu.VMEM((2, CHUNK, ROWS_PER_LINE * DIM), jnp.float32),
            pltpu.VMEM((WB_SIZE,), jnp.float32),
            pltpu.VMEM((2 * B_PER_W,), jnp.float32),
            pltpu.SemaphoreType.DMA,
        ],
        compiler_params=pltpu.CompilerParams(needs_layout_passes=False),
    )
    return f(a, p, emb_lines, wbb)


def kernel(a, p, embeddings, W, b):
    wb = jnp.broadcast_to(W.reshape(2 * DIM * 2, 1), (2 * DIM * 2, L))
    bb = jnp.broadcast_to(b.reshape(2, 1), (2, L))
    wbb = jnp.concatenate([wb.reshape(-1), bb.reshape(-1)])
    out_flat = _linear_model_sc(a.astype(jnp.int32), p.astype(jnp.int32),
                                embeddings, wbb)
    return out_flat.reshape(BATCH, 2)
